# Initial kernel scaffold; baseline (speedup 1.0000x reference)
import jax
import jax.numpy as jnp
from jax.experimental import pallas as pl


def kernel(x, edge_index, edge_attr, W_e, b_e, W_pre, b_pre, W_post, b_post, W_lin, b_lin):
    raise NotImplementedError("write your pallas kernel here")


# speedup vs baseline: 1.4940x; 1.4940x over previous
import math

import jax
import jax.numpy as jnp
from jax.experimental import pallas as pl
from jax.experimental.pallas import tpu as pltpu

_AVG_DEG_LOG = math.log(33.0)


def _prep_kernel(x_ref, w1_ref, w2_ref, c0_ref, a_ref, b_ref):
    xb = x_ref[...]
    a_ref[...] = (
        jnp.dot(xb, w1_ref[...], preferred_element_type=jnp.float32) + c0_ref[...]
    )
    b_ref[...] = jnp.dot(xb, w2_ref[...], preferred_element_type=jnp.float32)


def _edge_kernel(dst_ref, src_ref, ea_ref, ct_ref, a_ref, b_ref,
                 ssum_ref, ssq_ref, smax_ref, smin_ref, deg_ref, et_ref,
                 *, tile_e):
    @pl.when(pl.program_id(0) == 0)
    def _init():
        ssum_ref[...] = jnp.zeros_like(ssum_ref)
        ssq_ref[...] = jnp.zeros_like(ssq_ref)
        smax_ref[...] = jnp.full_like(smax_ref, -jnp.inf)
        smin_ref[...] = jnp.full_like(smin_ref, jnp.inf)
        deg_ref[...] = jnp.zeros_like(deg_ref)

    et_ref[...] = jnp.dot(ea_ref[...], ct_ref[...],
                          preferred_element_type=jnp.float32)

    def body(i, _):
        d = dst_ref[i]
        s = src_ref[i]
        m = (a_ref[pl.ds(d, 1), :] + b_ref[pl.ds(s, 1), :]
             + et_ref[pl.ds(i, 1), :])
        ssum_ref[pl.ds(d, 1), :] = ssum_ref[pl.ds(d, 1), :] + m
        ssq_ref[pl.ds(d, 1), :] = ssq_ref[pl.ds(d, 1), :] + m * m
        smax_ref[pl.ds(d, 1), :] = jnp.maximum(smax_ref[pl.ds(d, 1), :], m)
        smin_ref[pl.ds(d, 1), :] = jnp.minimum(smin_ref[pl.ds(d, 1), :], m)
        deg_ref[pl.ds(d, 1), :] = deg_ref[pl.ds(d, 1), :] + 1.0
        return 0

    jax.lax.fori_loop(0, tile_e, body, 0)


def _node_kernel(x_ref, ssum_ref, ssq_ref, smax_ref, smin_ref, deg_ref,
                 wpost_ref, bpost_ref, wlin_ref, blin_ref, out_ref):
    degc = jnp.maximum(deg_ref[...], 1.0)
    mean = ssum_ref[...] / degc
    meansq = ssq_ref[...] / degc
    var = meansq - mean * mean
    std = jnp.sqrt(jnp.maximum(var, 0.0) + 1e-5)
    mx = smax_ref[...]
    mx = jnp.where(jnp.isfinite(mx), mx, 0.0)
    mn = smin_ref[...]
    mn = jnp.where(jnp.isfinite(mn), mn, 0.0)
    agg = jnp.concatenate([mean, mn, mx, std], axis=-1)
    logdeg = jnp.log(degc + 1.0)
    amp = agg * (logdeg / _AVG_DEG_LOG)
    att = agg * (_AVG_DEG_LOG / logdeg)
    h = jnp.concatenate([x_ref[...], agg, amp, att], axis=-1)
    o = (jnp.dot(h, wpost_ref[...], preferred_element_type=jnp.float32)
         + bpost_ref[...])
    out_ref[...] = (
        jnp.dot(o, wlin_ref[...], preferred_element_type=jnp.float32)
        + blin_ref[...]
    )


def kernel(x, edge_index, edge_attr, W_e, b_e, W_pre, b_pre,
           W_post, b_post, W_lin, b_lin):
    n, f = x.shape
    e_cnt = edge_index.shape[1]
    ed = edge_attr.shape[1]
    f32 = jnp.float32

    tile_n = 1000 if n % 1000 == 0 else n
    tile_e = 512 if e_cnt % 512 == 0 else e_cnt

    wpre_t = W_pre.T
    w1t = wpre_t[:f]
    w2t = wpre_t[f:2 * f]
    w3t = wpre_t[2 * f:]
    ct = W_e.T @ w3t
    c0 = (b_pre + b_e @ w3t).reshape(1, f)

    src = edge_index[0]
    dst = edge_index[1]

    a, b = pl.pallas_call(
        _prep_kernel,
        grid=(n // tile_n,),
        in_specs=[
            pl.BlockSpec((tile_n, f), lambda i: (i, 0)),
            pl.BlockSpec((f, f), lambda i: (0, 0)),
            pl.BlockSpec((f, f), lambda i: (0, 0)),
            pl.BlockSpec((1, f), lambda i: (0, 0)),
        ],
        out_specs=[
            pl.BlockSpec((tile_n, f), lambda i: (i, 0)),
            pl.BlockSpec((tile_n, f), lambda i: (i, 0)),
        ],
        out_shape=[
            jax.ShapeDtypeStruct((n, f), f32),
            jax.ShapeDtypeStruct((n, f), f32),
        ],
    )(x, w1t, w2t, c0)

    import functools
    ssum, ssq, smax, smin, deg = pl.pallas_call(
        functools.partial(_edge_kernel, tile_e=tile_e),
        grid=(e_cnt // tile_e,),
        in_specs=[
            pl.BlockSpec((tile_e,), lambda i: (i,), memory_space=pltpu.SMEM),
            pl.BlockSpec((tile_e,), lambda i: (i,), memory_space=pltpu.SMEM),
            pl.BlockSpec((tile_e, ed), lambda i: (i, 0)),
            pl.BlockSpec((ed, f), lambda i: (0, 0)),
            pl.BlockSpec((n, f), lambda i: (0, 0)),
            pl.BlockSpec((n, f), lambda i: (0, 0)),
        ],
        out_specs=[
            pl.BlockSpec((n, f), lambda i: (0, 0)),
            pl.BlockSpec((n, f), lambda i: (0, 0)),
            pl.BlockSpec((n, f), lambda i: (0, 0)),
            pl.BlockSpec((n, f), lambda i: (0, 0)),
            pl.BlockSpec((n, 1), lambda i: (0, 0)),
        ],
        out_shape=[
            jax.ShapeDtypeStruct((n, f), f32),
            jax.ShapeDtypeStruct((n, f), f32),
            jax.ShapeDtypeStruct((n, f), f32),
            jax.ShapeDtypeStruct((n, f), f32),
            jax.ShapeDtypeStruct((n, 1), f32),
        ],
        scratch_shapes=[pltpu.VMEM((tile_e, f), f32)],
        compiler_params=pltpu.CompilerParams(
            vmem_limit_bytes=100 * 1024 * 1024,
        ),
    )(dst, src, edge_attr, ct, a, b)

    out = pl.pallas_call(
        _node_kernel,
        grid=(n // tile_n,),
        in_specs=[
            pl.BlockSpec((tile_n, f), lambda i: (i, 0)),
            pl.BlockSpec((tile_n, f), lambda i: (i, 0)),
            pl.BlockSpec((tile_n, f), lambda i: (i, 0)),
            pl.BlockSpec((tile_n, f), lambda i: (i, 0)),
            pl.BlockSpec((tile_n, f), lambda i: (i, 0)),
            pl.BlockSpec((tile_n, 1), lambda i: (i, 0)),
            pl.BlockSpec((13 * f, f), lambda i: (0, 0)),
            pl.BlockSpec((1, f), lambda i: (0, 0)),
            pl.BlockSpec((f, f), lambda i: (0, 0)),
            pl.BlockSpec((1, f), lambda i: (0, 0)),
        ],
        out_specs=pl.BlockSpec((tile_n, f), lambda i: (i, 0)),
        out_shape=jax.ShapeDtypeStruct((n, f), f32),
    )(x, ssum, ssq, smax, smin, deg,
      W_post.T, b_post.reshape(1, f), W_lin.T, b_lin.reshape(1, f))
    return out


# speedup vs baseline: 2.3427x; 1.5680x over previous
import math

import jax
import jax.numpy as jnp
from jax.experimental import pallas as pl
from jax.experimental.pallas import tpu as pltpu

_AVG_DEG_LOG = math.log(33.0)


def _prep_kernel(x_ref, w1_ref, w2_ref, c0_ref, a_ref, b_ref):
    xb = x_ref[...]
    a_ref[...] = (
        jnp.dot(xb, w1_ref[...], preferred_element_type=jnp.float32) + c0_ref[...]
    )
    b_ref[...] = jnp.dot(xb, w2_ref[...], preferred_element_type=jnp.float32)


def _edge_kernel(dst_ref, src_ref, ea_ref, ct_ref, a_ref, b_ref,
                 ssum_ref, ssq_ref, smax_ref, smin_ref, deg_ref, et_ref,
                 *, tile_e):
    @pl.when(pl.program_id(0) == 0)
    def _init():
        ssum_ref[...] = jnp.zeros_like(ssum_ref)
        ssq_ref[...] = jnp.zeros_like(ssq_ref)
        smax_ref[...] = jnp.full_like(smax_ref, -jnp.inf)
        smin_ref[...] = jnp.full_like(smin_ref, jnp.inf)
        deg_ref[...] = jnp.zeros_like(deg_ref)

    et_ref[...] = jnp.dot(ea_ref[...], ct_ref[...],
                          preferred_element_type=jnp.float32)

    def body(i, _):
        d = dst_ref[i]
        s = src_ref[i]
        m = (a_ref[pl.ds(d, 1), :] + b_ref[pl.ds(s, 1), :]
             + et_ref[pl.ds(i, 1), :])
        ssum_ref[pl.ds(d, 1), :] = ssum_ref[pl.ds(d, 1), :] + m
        ssq_ref[pl.ds(d, 1), :] = ssq_ref[pl.ds(d, 1), :] + m * m
        smax_ref[pl.ds(d, 1), :] = jnp.maximum(smax_ref[pl.ds(d, 1), :], m)
        smin_ref[pl.ds(d, 1), :] = jnp.minimum(smin_ref[pl.ds(d, 1), :], m)
        deg_ref[pl.ds(d, 1), :] = deg_ref[pl.ds(d, 1), :] + 1.0
        return 0

    jax.lax.fori_loop(0, tile_e, body, 0, unroll=4)


def _node_kernel(x_ref, ssum_ref, ssq_ref, smax_ref, smin_ref, deg_ref,
                 wpost_ref, bpost_ref, wlin_ref, blin_ref, out_ref):
    degc = jnp.maximum(deg_ref[...], 1.0)
    mean = ssum_ref[...] / degc
    meansq = ssq_ref[...] / degc
    var = meansq - mean * mean
    std = jnp.sqrt(jnp.maximum(var, 0.0) + 1e-5)
    mx = smax_ref[...]
    mx = jnp.where(jnp.isfinite(mx), mx, 0.0)
    mn = smin_ref[...]
    mn = jnp.where(jnp.isfinite(mn), mn, 0.0)
    agg = jnp.concatenate([mean, mn, mx, std], axis=-1)
    logdeg = jnp.log(degc + 1.0)
    amp = agg * (logdeg / _AVG_DEG_LOG)
    att = agg * (_AVG_DEG_LOG / logdeg)
    h = jnp.concatenate([x_ref[...], agg, amp, att], axis=-1)
    o = (jnp.dot(h, wpost_ref[...], preferred_element_type=jnp.float32)
         + bpost_ref[...])
    out_ref[...] = (
        jnp.dot(o, wlin_ref[...], preferred_element_type=jnp.float32)
        + blin_ref[...]
    )


def kernel(x, edge_index, edge_attr, W_e, b_e, W_pre, b_pre,
           W_post, b_post, W_lin, b_lin):
    n, f = x.shape
    e_cnt = edge_index.shape[1]
    ed = edge_attr.shape[1]
    f32 = jnp.float32

    tile_n = 1000 if n % 1000 == 0 else n
    tile_e = 512 if e_cnt % 512 == 0 else e_cnt

    wpre_t = W_pre.T
    w1t = wpre_t[:f]
    w2t = wpre_t[f:2 * f]
    w3t = wpre_t[2 * f:]
    ct = W_e.T @ w3t
    c0 = (b_pre + b_e @ w3t).reshape(1, f)

    src = edge_index[0]
    dst = edge_index[1]

    a, b = pl.pallas_call(
        _prep_kernel,
        grid=(n // tile_n,),
        in_specs=[
            pl.BlockSpec((tile_n, f), lambda i: (i, 0)),
            pl.BlockSpec((f, f), lambda i: (0, 0)),
            pl.BlockSpec((f, f), lambda i: (0, 0)),
            pl.BlockSpec((1, f), lambda i: (0, 0)),
        ],
        out_specs=[
            pl.BlockSpec((tile_n, f), lambda i: (i, 0)),
            pl.BlockSpec((tile_n, f), lambda i: (i, 0)),
        ],
        out_shape=[
            jax.ShapeDtypeStruct((n, f), f32),
            jax.ShapeDtypeStruct((n, f), f32),
        ],
    )(x, w1t, w2t, c0)

    import functools
    ssum, ssq, smax, smin, deg = pl.pallas_call(
        functools.partial(_edge_kernel, tile_e=tile_e),
        grid=(e_cnt // tile_e,),
        in_specs=[
            pl.BlockSpec((tile_e,), lambda i: (i,), memory_space=pltpu.SMEM),
            pl.BlockSpec((tile_e,), lambda i: (i,), memory_space=pltpu.SMEM),
            pl.BlockSpec((tile_e, ed), lambda i: (i, 0)),
            pl.BlockSpec((ed, f), lambda i: (0, 0)),
            pl.BlockSpec((n, f), lambda i: (0, 0)),
            pl.BlockSpec((n, f), lambda i: (0, 0)),
        ],
        out_specs=[
            pl.BlockSpec((n, f), lambda i: (0, 0)),
            pl.BlockSpec((n, f), lambda i: (0, 0)),
            pl.BlockSpec((n, f), lambda i: (0, 0)),
            pl.BlockSpec((n, f), lambda i: (0, 0)),
            pl.BlockSpec((n, 1), lambda i: (0, 0)),
        ],
        out_shape=[
            jax.ShapeDtypeStruct((n, f), f32),
            jax.ShapeDtypeStruct((n, f), f32),
            jax.ShapeDtypeStruct((n, f), f32),
            jax.ShapeDtypeStruct((n, f), f32),
            jax.ShapeDtypeStruct((n, 1), f32),
        ],
        scratch_shapes=[pltpu.VMEM((tile_e, f), f32)],
        compiler_params=pltpu.CompilerParams(
            vmem_limit_bytes=100 * 1024 * 1024,
        ),
    )(dst, src, edge_attr, ct, a, b)

    out = pl.pallas_call(
        _node_kernel,
        grid=(n // tile_n,),
        in_specs=[
            pl.BlockSpec((tile_n, f), lambda i: (i, 0)),
            pl.BlockSpec((tile_n, f), lambda i: (i, 0)),
            pl.BlockSpec((tile_n, f), lambda i: (i, 0)),
            pl.BlockSpec((tile_n, f), lambda i: (i, 0)),
            pl.BlockSpec((tile_n, f), lambda i: (i, 0)),
            pl.BlockSpec((tile_n, 1), lambda i: (i, 0)),
            pl.BlockSpec((13 * f, f), lambda i: (0, 0)),
            pl.BlockSpec((1, f), lambda i: (0, 0)),
            pl.BlockSpec((f, f), lambda i: (0, 0)),
            pl.BlockSpec((1, f), lambda i: (0, 0)),
        ],
        out_specs=pl.BlockSpec((tile_n, f), lambda i: (i, 0)),
        out_shape=jax.ShapeDtypeStruct((n, f), f32),
    )(x, ssum, ssq, smax, smin, deg,
      W_post.T, b_post.reshape(1, f), W_lin.T, b_lin.reshape(1, f))
    return out


# speedup vs baseline: 2.5099x; 1.0714x over previous
import math

import jax
import jax.numpy as jnp
from jax.experimental import pallas as pl
from jax.experimental.pallas import tpu as pltpu

_AVG_DEG_LOG = math.log(33.0)


def _prep_kernel(x_ref, w1_ref, w2_ref, c0_ref, a_ref, b_ref):
    xb = x_ref[...]
    a_ref[...] = (
        jnp.dot(xb, w1_ref[...], preferred_element_type=jnp.float32) + c0_ref[...]
    )
    b_ref[...] = jnp.dot(xb, w2_ref[...], preferred_element_type=jnp.float32)


def _edge_kernel(dst_ref, src_ref, ea_ref, ct_ref, a_ref, b_ref,
                 ssum_ref, ssq_ref, smax_ref, smin_ref, deg_ref, et_ref,
                 *, tile_e):
    @pl.when(pl.program_id(0) == 0)
    def _init():
        ssum_ref[...] = jnp.zeros_like(ssum_ref)
        ssq_ref[...] = jnp.zeros_like(ssq_ref)
        smax_ref[...] = jnp.full_like(smax_ref, -jnp.inf)
        smin_ref[...] = jnp.full_like(smin_ref, jnp.inf)
        deg_ref[...] = jnp.zeros_like(deg_ref)

    et_ref[...] = jnp.dot(ea_ref[...], ct_ref[...],
                          preferred_element_type=jnp.float32)

    def body(i, _):
        d = dst_ref[i]
        s = src_ref[i]
        m = (a_ref[pl.ds(d, 1), :] + b_ref[pl.ds(s, 1), :]
             + et_ref[pl.ds(i, 1), :])
        ssum_ref[pl.ds(d, 1), :] = ssum_ref[pl.ds(d, 1), :] + m
        ssq_ref[pl.ds(d, 1), :] = ssq_ref[pl.ds(d, 1), :] + m * m
        smax_ref[pl.ds(d, 1), :] = jnp.maximum(smax_ref[pl.ds(d, 1), :], m)
        smin_ref[pl.ds(d, 1), :] = jnp.minimum(smin_ref[pl.ds(d, 1), :], m)
        deg_ref[pl.ds(d, 1), :] = deg_ref[pl.ds(d, 1), :] + 1.0
        return 0

    jax.lax.fori_loop(0, tile_e, body, 0, unroll=8)


def _node_kernel(x_ref, ssum_ref, ssq_ref, smax_ref, smin_ref, deg_ref,
                 wpost_ref, bpost_ref, wlin_ref, blin_ref, out_ref):
    degc = jnp.maximum(deg_ref[...], 1.0)
    mean = ssum_ref[...] / degc
    meansq = ssq_ref[...] / degc
    var = meansq - mean * mean
    std = jnp.sqrt(jnp.maximum(var, 0.0) + 1e-5)
    mx = smax_ref[...]
    mx = jnp.where(jnp.isfinite(mx), mx, 0.0)
    mn = smin_ref[...]
    mn = jnp.where(jnp.isfinite(mn), mn, 0.0)
    agg = jnp.concatenate([mean, mn, mx, std], axis=-1)
    logdeg = jnp.log(degc + 1.0)
    amp = agg * (logdeg / _AVG_DEG_LOG)
    att = agg * (_AVG_DEG_LOG / logdeg)
    h = jnp.concatenate([x_ref[...], agg, amp, att], axis=-1)
    o = (jnp.dot(h, wpost_ref[...], preferred_element_type=jnp.float32)
         + bpost_ref[...])
    out_ref[...] = (
        jnp.dot(o, wlin_ref[...], preferred_element_type=jnp.float32)
        + blin_ref[...]
    )


def kernel(x, edge_index, edge_attr, W_e, b_e, W_pre, b_pre,
           W_post, b_post, W_lin, b_lin):
    n, f = x.shape
    e_cnt = edge_index.shape[1]
    ed = edge_attr.shape[1]
    f32 = jnp.float32

    tile_n = 1000 if n % 1000 == 0 else n
    tile_e = 512 if e_cnt % 512 == 0 else e_cnt

    wpre_t = W_pre.T
    w1t = wpre_t[:f]
    w2t = wpre_t[f:2 * f]
    w3t = wpre_t[2 * f:]
    ct = W_e.T @ w3t
    c0 = (b_pre + b_e @ w3t).reshape(1, f)

    src = edge_index[0]
    dst = edge_index[1]

    a, b = pl.pallas_call(
        _prep_kernel,
        grid=(n // tile_n,),
        in_specs=[
            pl.BlockSpec((tile_n, f), lambda i: (i, 0)),
            pl.BlockSpec((f, f), lambda i: (0, 0)),
            pl.BlockSpec((f, f), lambda i: (0, 0)),
            pl.BlockSpec((1, f), lambda i: (0, 0)),
        ],
        out_specs=[
            pl.BlockSpec((tile_n, f), lambda i: (i, 0)),
            pl.BlockSpec((tile_n, f), lambda i: (i, 0)),
        ],
        out_shape=[
            jax.ShapeDtypeStruct((n, f), f32),
            jax.ShapeDtypeStruct((n, f), f32),
        ],
    )(x, w1t, w2t, c0)

    import functools
    ssum, ssq, smax, smin, deg = pl.pallas_call(
        functools.partial(_edge_kernel, tile_e=tile_e),
        grid=(e_cnt // tile_e,),
        in_specs=[
            pl.BlockSpec((tile_e,), lambda i: (i,), memory_space=pltpu.SMEM),
            pl.BlockSpec((tile_e,), lambda i: (i,), memory_space=pltpu.SMEM),
            pl.BlockSpec((tile_e, ed), lambda i: (i, 0)),
            pl.BlockSpec((ed, f), lambda i: (0, 0)),
            pl.BlockSpec((n, f), lambda i: (0, 0)),
            pl.BlockSpec((n, f), lambda i: (0, 0)),
        ],
        out_specs=[
            pl.BlockSpec((n, f), lambda i: (0, 0)),
            pl.BlockSpec((n, f), lambda i: (0, 0)),
            pl.BlockSpec((n, f), lambda i: (0, 0)),
            pl.BlockSpec((n, f), lambda i: (0, 0)),
            pl.BlockSpec((n, 1), lambda i: (0, 0)),
        ],
        out_shape=[
            jax.ShapeDtypeStruct((n, f), f32),
            jax.ShapeDtypeStruct((n, f), f32),
            jax.ShapeDtypeStruct((n, f), f32),
            jax.ShapeDtypeStruct((n, f), f32),
            jax.ShapeDtypeStruct((n, 1), f32),
        ],
        scratch_shapes=[pltpu.VMEM((tile_e, f), f32)],
        compiler_params=pltpu.CompilerParams(
            vmem_limit_bytes=100 * 1024 * 1024,
        ),
    )(dst, src, edge_attr, ct, a, b)

    out = pl.pallas_call(
        _node_kernel,
        grid=(n // tile_n,),
        in_specs=[
            pl.BlockSpec((tile_n, f), lambda i: (i, 0)),
            pl.BlockSpec((tile_n, f), lambda i: (i, 0)),
            pl.BlockSpec((tile_n, f), lambda i: (i, 0)),
            pl.BlockSpec((tile_n, f), lambda i: (i, 0)),
            pl.BlockSpec((tile_n, f), lambda i: (i, 0)),
            pl.BlockSpec((tile_n, 1), lambda i: (i, 0)),
            pl.BlockSpec((13 * f, f), lambda i: (0, 0)),
            pl.BlockSpec((1, f), lambda i: (0, 0)),
            pl.BlockSpec((f, f), lambda i: (0, 0)),
            pl.BlockSpec((1, f), lambda i: (0, 0)),
        ],
        out_specs=pl.BlockSpec((tile_n, f), lambda i: (i, 0)),
        out_shape=jax.ShapeDtypeStruct((n, f), f32),
    )(x, ssum, ssq, smax, smin, deg,
      W_post.T, b_post.reshape(1, f), W_lin.T, b_lin.reshape(1, f))
    return out


# speedup vs baseline: 2.5746x; 1.0258x over previous
import math

import jax
import jax.numpy as jnp
from jax.experimental import pallas as pl
from jax.experimental.pallas import tpu as pltpu

_AVG_DEG_LOG = math.log(33.0)


def _prep_kernel(x_ref, w1_ref, w2_ref, c0_ref, a_ref, b_ref):
    xb = x_ref[...]
    a_ref[...] = (
        jnp.dot(xb, w1_ref[...], preferred_element_type=jnp.float32) + c0_ref[...]
    )
    b_ref[...] = jnp.dot(xb, w2_ref[...], preferred_element_type=jnp.float32)


def _edge_kernel(dst_ref, src_ref, ea_ref, ct_ref, a_ref, b_ref,
                 ssum_ref, ssq_ref, smax_ref, smin_ref, deg_ref, et_ref,
                 *, tile_e):
    @pl.when(pl.program_id(0) == 0)
    def _init():
        ssum_ref[...] = jnp.zeros_like(ssum_ref)
        ssq_ref[...] = jnp.zeros_like(ssq_ref)
        smax_ref[...] = jnp.full_like(smax_ref, -jnp.inf)
        smin_ref[...] = jnp.full_like(smin_ref, jnp.inf)
        deg_ref[...] = jnp.zeros_like(deg_ref)

    et_ref[...] = jnp.dot(ea_ref[...], ct_ref[...],
                          preferred_element_type=jnp.float32)

    def body(i, _):
        d = dst_ref[i]
        s = src_ref[i]
        m = (a_ref[pl.ds(d, 1), :] + b_ref[pl.ds(s, 1), :]
             + et_ref[pl.ds(i, 1), :])
        ssum_ref[pl.ds(d, 1), :] = ssum_ref[pl.ds(d, 1), :] + m
        ssq_ref[pl.ds(d, 1), :] = ssq_ref[pl.ds(d, 1), :] + m * m
        smax_ref[pl.ds(d, 1), :] = jnp.maximum(smax_ref[pl.ds(d, 1), :], m)
        smin_ref[pl.ds(d, 1), :] = jnp.minimum(smin_ref[pl.ds(d, 1), :], m)
        deg_ref[pl.ds(d, 1), :] = deg_ref[pl.ds(d, 1), :] + 1.0
        return 0

    jax.lax.fori_loop(0, tile_e, body, 0, unroll=16)


def _node_kernel(x_ref, ssum_ref, ssq_ref, smax_ref, smin_ref, deg_ref,
                 wpost_ref, bpost_ref, wlin_ref, blin_ref, out_ref):
    degc = jnp.maximum(deg_ref[...], 1.0)
    mean = ssum_ref[...] / degc
    meansq = ssq_ref[...] / degc
    var = meansq - mean * mean
    std = jnp.sqrt(jnp.maximum(var, 0.0) + 1e-5)
    mx = smax_ref[...]
    mx = jnp.where(jnp.isfinite(mx), mx, 0.0)
    mn = smin_ref[...]
    mn = jnp.where(jnp.isfinite(mn), mn, 0.0)
    agg = jnp.concatenate([mean, mn, mx, std], axis=-1)
    logdeg = jnp.log(degc + 1.0)
    amp = agg * (logdeg / _AVG_DEG_LOG)
    att = agg * (_AVG_DEG_LOG / logdeg)
    h = jnp.concatenate([x_ref[...], agg, amp, att], axis=-1)
    o = (jnp.dot(h, wpost_ref[...], preferred_element_type=jnp.float32)
         + bpost_ref[...])
    out_ref[...] = (
        jnp.dot(o, wlin_ref[...], preferred_element_type=jnp.float32)
        + blin_ref[...]
    )


def kernel(x, edge_index, edge_attr, W_e, b_e, W_pre, b_pre,
           W_post, b_post, W_lin, b_lin):
    n, f = x.shape
    e_cnt = edge_index.shape[1]
    ed = edge_attr.shape[1]
    f32 = jnp.float32

    tile_n = 1000 if n % 1000 == 0 else n
    tile_e = 512 if e_cnt % 512 == 0 else e_cnt

    wpre_t = W_pre.T
    w1t = wpre_t[:f]
    w2t = wpre_t[f:2 * f]
    w3t = wpre_t[2 * f:]
    ct = W_e.T @ w3t
    c0 = (b_pre + b_e @ w3t).reshape(1, f)

    src = edge_index[0]
    dst = edge_index[1]

    a, b = pl.pallas_call(
        _prep_kernel,
        grid=(n // tile_n,),
        in_specs=[
            pl.BlockSpec((tile_n, f), lambda i: (i, 0)),
            pl.BlockSpec((f, f), lambda i: (0, 0)),
            pl.BlockSpec((f, f), lambda i: (0, 0)),
            pl.BlockSpec((1, f), lambda i: (0, 0)),
        ],
        out_specs=[
            pl.BlockSpec((tile_n, f), lambda i: (i, 0)),
            pl.BlockSpec((tile_n, f), lambda i: (i, 0)),
        ],
        out_shape=[
            jax.ShapeDtypeStruct((n, f), f32),
            jax.ShapeDtypeStruct((n, f), f32),
        ],
    )(x, w1t, w2t, c0)

    import functools
    ssum, ssq, smax, smin, deg = pl.pallas_call(
        functools.partial(_edge_kernel, tile_e=tile_e),
        grid=(e_cnt // tile_e,),
        in_specs=[
            pl.BlockSpec((tile_e,), lambda i: (i,), memory_space=pltpu.SMEM),
            pl.BlockSpec((tile_e,), lambda i: (i,), memory_space=pltpu.SMEM),
            pl.BlockSpec((tile_e, ed), lambda i: (i, 0)),
            pl.BlockSpec((ed, f), lambda i: (0, 0)),
            pl.BlockSpec((n, f), lambda i: (0, 0)),
            pl.BlockSpec((n, f), lambda i: (0, 0)),
        ],
        out_specs=[
            pl.BlockSpec((n, f), lambda i: (0, 0)),
            pl.BlockSpec((n, f), lambda i: (0, 0)),
            pl.BlockSpec((n, f), lambda i: (0, 0)),
            pl.BlockSpec((n, f), lambda i: (0, 0)),
            pl.BlockSpec((n, 1), lambda i: (0, 0)),
        ],
        out_shape=[
            jax.ShapeDtypeStruct((n, f), f32),
            jax.ShapeDtypeStruct((n, f), f32),
            jax.ShapeDtypeStruct((n, f), f32),
            jax.ShapeDtypeStruct((n, f), f32),
            jax.ShapeDtypeStruct((n, 1), f32),
        ],
        scratch_shapes=[pltpu.VMEM((tile_e, f), f32)],
        compiler_params=pltpu.CompilerParams(
            vmem_limit_bytes=100 * 1024 * 1024,
        ),
    )(dst, src, edge_attr, ct, a, b)

    out = pl.pallas_call(
        _node_kernel,
        grid=(n // tile_n,),
        in_specs=[
            pl.BlockSpec((tile_n, f), lambda i: (i, 0)),
            pl.BlockSpec((tile_n, f), lambda i: (i, 0)),
            pl.BlockSpec((tile_n, f), lambda i: (i, 0)),
            pl.BlockSpec((tile_n, f), lambda i: (i, 0)),
            pl.BlockSpec((tile_n, f), lambda i: (i, 0)),
            pl.BlockSpec((tile_n, 1), lambda i: (i, 0)),
            pl.BlockSpec((13 * f, f), lambda i: (0, 0)),
            pl.BlockSpec((1, f), lambda i: (0, 0)),
            pl.BlockSpec((f, f), lambda i: (0, 0)),
            pl.BlockSpec((1, f), lambda i: (0, 0)),
        ],
        out_specs=pl.BlockSpec((tile_n, f), lambda i: (i, 0)),
        out_shape=jax.ShapeDtypeStruct((n, f), f32),
    )(x, ssum, ssq, smax, smin, deg,
      W_post.T, b_post.reshape(1, f), W_lin.T, b_lin.reshape(1, f))
    return out
